# baseline (speedup 1.0000x reference)
import functools

import numpy as np
import jax
import jax.numpy as jnp
from jax.experimental import pallas as pl
from jax.experimental.pallas import tpu as pltpu
from jax.sharding import Mesh, PartitionSpec as P_

_N = 4096


def _dft_mats():
    i = np.arange(_N)
    prod = (i[:, None].astype(np.int64) * i[None, :]) % _N
    theta = prod.astype(np.float64) * (2.0 * np.pi / _N)
    sign = np.where(i % 2 == 0, 1.0, -1.0)[:, None]
    c = sign * np.cos(theta) / _N
    s = sign * np.sin(theta) / _N
    bf = jnp.bfloat16
    return (c.astype(bf), s.astype(bf),
            np.ascontiguousarray(c.T).astype(bf),
            np.ascontiguousarray(s.T).astype(bf))


_C, _S, _CT, _ST = _dft_mats()

_BM = 512
_BN = 512
_PW_ROWS = 256
_VMEM = 60 * 1024 * 1024


def _pointwise_body(xr_ref, xi_ref, mk_ref, pk_ref, or_ref, oi_ref):
    xr = xr_ref[...]
    xi = xi_ref[...]
    mag = jnp.sqrt(xr * xr + xi * xi) * mk_ref[...]
    ph = jnp.arctan2(xi, xr) * pk_ref[...]
    or_ref[...] = (mag * jnp.cos(ph)).astype(jnp.bfloat16)
    oi_ref[...] = (mag * jnp.sin(ph)).astype(jnp.bfloat16)


def _stage1_body(c_ref, s_ref, xr_ref, xi_ref, p_ref, q_ref):
    c = c_ref[...]
    s = s_ref[...]
    xr = xr_ref[...]
    xi = xi_ref[...]
    p_ref[...] = (jnp.dot(c, xr, preferred_element_type=jnp.float32)
                  - jnp.dot(s, xi, preferred_element_type=jnp.float32)
                  ).astype(jnp.bfloat16)
    q_ref[...] = (jnp.dot(c, xi, preferred_element_type=jnp.float32)
                  + jnp.dot(s, xr, preferred_element_type=jnp.float32)
                  ).astype(jnp.bfloat16)


def _stage2_body(p_ref, q_ref, ct_ref, st_ref, o_ref):
    o_ref[...] = (jnp.dot(p_ref[...], ct_ref[...], preferred_element_type=jnp.float32)
                  - jnp.dot(q_ref[...], st_ref[...], preferred_element_type=jnp.float32))


def _pointwise_call(xr, xi, mk, pk):
    rows = xr.shape[0]
    spec = pl.BlockSpec((_PW_ROWS, _N), lambda i: (i, 0))
    return pl.pallas_call(
        _pointwise_body,
        grid=(rows // _PW_ROWS,),
        in_specs=[spec] * 4,
        out_specs=[spec] * 2,
        out_shape=[jax.ShapeDtypeStruct((rows, _N), jnp.bfloat16)] * 2,
        compiler_params=pltpu.CompilerParams(
            dimension_semantics=("arbitrary",),
            vmem_limit_bytes=_VMEM,
        ),
    )(xr, xi, mk, pk)


def _stage1_call(c, s, Xr, Xi):
    rows = c.shape[0]
    lhs_spec = pl.BlockSpec((_BM, _N), lambda i, j: (i, 0))
    rhs_spec = pl.BlockSpec((_N, _BN), lambda i, j: (0, j))
    out_spec = pl.BlockSpec((_BM, _BN), lambda i, j: (i, j))
    return pl.pallas_call(
        _stage1_body,
        grid=(rows // _BM, _N // _BN),
        in_specs=[lhs_spec, lhs_spec, rhs_spec, rhs_spec],
        out_specs=[out_spec, out_spec],
        out_shape=[jax.ShapeDtypeStruct((rows, _N), jnp.bfloat16)] * 2,
        compiler_params=pltpu.CompilerParams(
            dimension_semantics=("arbitrary", "arbitrary"),
            vmem_limit_bytes=_VMEM,
        ),
    )(c, s, Xr, Xi)


def _stage2_call(p, q, ct, st):
    rows = p.shape[0]
    lhs_spec = pl.BlockSpec((_BM, _N), lambda i, j: (i, 0))
    rhs_spec = pl.BlockSpec((_N, _BN), lambda i, j: (0, j))
    out_spec = pl.BlockSpec((_BM, _BN), lambda i, j: (i, j))
    return pl.pallas_call(
        _stage2_body,
        grid=(rows // _BM, _N // _BN),
        in_specs=[lhs_spec, lhs_spec, rhs_spec, rhs_spec],
        out_specs=out_spec,
        out_shape=jax.ShapeDtypeStruct((rows, _N), jnp.float32),
        compiler_params=pltpu.CompilerParams(
            dimension_semantics=("arbitrary", "arbitrary"),
            vmem_limit_bytes=_VMEM,
        ),
    )(p, q, ct, st)


def _local_body(xr, xi, mk, pk, c, s, ct, st):
    Xr_h, Xi_h = _pointwise_call(xr, xi, mk, pk)
    Xr = jax.lax.all_gather(Xr_h, "x", axis=0, tiled=True)
    Xi = jax.lax.all_gather(Xi_h, "x", axis=0, tiled=True)
    p, q = _stage1_call(c, s, Xr, Xi)
    return _stage2_call(p, q, ct, st)


def _single_device(xr, xi, mk, pk):
    Xr, Xi = _pointwise_call(xr, xi, mk, pk)
    p, q = _stage1_call(_C, _S, Xr, Xi)
    return _stage2_call(p, q, _CT, _ST)


@jax.jit
def kernel(x_real, x_imag, magnitude_kernel, phase_kernel):
    xr = x_real.reshape(_N, _N)
    xi = x_imag.reshape(_N, _N)
    mk = magnitude_kernel.reshape(_N, _N)
    pk = phase_kernel.reshape(_N, _N)

    devs = jax.devices()
    if len(devs) >= 2:
        mesh = Mesh(np.array(devs[:2]), ("x",))
        fn = jax.shard_map(
            _local_body,
            mesh=mesh,
            in_specs=(P_("x"), P_("x"), P_("x"), P_("x"),
                      P_("x"), P_("x"), P_(None, None), P_(None, None)),
            out_specs=P_("x"),
            check_vma=False,
        )
        out = fn(xr, xi, mk, pk, _C, _S, _CT, _ST)
        out = jax.device_put(out, devs[0])
    else:
        out = _single_device(xr, xi, mk, pk)

    return out.reshape(1, _N, _N)
